# baseline (speedup 1.0000x reference)
import jax
import jax.numpy as jnp
from jax import lax
from jax.experimental import pallas as pl
from jax.experimental.pallas import tpu as pltpu
from jax.experimental.pallas import tpu_sc as plsc

B = 16384
D = 64
NE = 1000000
NR = 1000
NC = 2
NS = 16
L = 16
NW = NC * NS
BPW = B // NW
C = 128
NCHUNK = BPW // C
G = C // L


def _body(heads, rels, tails, eh, et, rw, riw, out,
          idx_h, idx_r, idx_t, relv, b0, b1, qa, qb, fwd_v, out_v, sem):
    cid = lax.axis_index("c")
    sid = lax.axis_index("s")
    wid = sid * NC + cid
    base = wid * BPW

    pltpu.sync_copy(heads.at[pl.ds(base, BPW)], idx_h)
    pltpu.sync_copy(rels.at[pl.ds(base, BPW)], idx_r)
    pltpu.sync_copy(tails.at[pl.ds(base, BPW)], idx_t)

    lane = lax.iota(jnp.int32, L)

    def run_pass(rel_flat, ia_ref, ib_ref, emit):
        pltpu.sync_copy(rel_flat, relv)

        for c in range(NCHUNK):
            for g in range(G):
                va = ia_ref[pl.ds(c * C + g * L, L)]
                vb = ib_ref[pl.ds(c * C + g * L, L)]
                qa[pl.ds(g * L, L)] = va >> 2
                qb[pl.ds(g * L, L)] = vb >> 2
            cp0 = pltpu.async_copy(eh.at[qa], b0, sem)
            cp1 = pltpu.async_copy(et.at[qb], b1, sem)
            cp0.wait()
            cp1.wait()

            himask = jnp.full((L,), -65536, jnp.int32)

            for g in range(G):
                va = ia_ref[pl.ds(c * C + g * L, L)]
                vb = ib_ref[pl.ds(c * C + g * L, L)]
                rowv = lane + g * L
                offa = (va & 3) * (D // 2)
                offb = (vb & 3) * (D // 2)
                q_vec = idx_r[pl.ds(c * C + g * L, L)]

                def wstep(w, acc, rowv=rowv, offa=offa, offb=offb,
                          q_vec=q_vec):
                    wa = plsc.load_gather(b0, [rowv, offa + w])
                    wb = plsc.load_gather(b1, [rowv, offb + w])
                    a0 = plsc.bitcast(wa << 16, jnp.float32)
                    b0v = plsc.bitcast(wb << 16, jnp.float32)
                    a1 = plsc.bitcast(wa & himask, jnp.float32)
                    b1v = plsc.bitcast(wb & himask, jnp.float32)
                    r0 = plsc.load_gather(relv, [q_vec + (2 * w) * NR])
                    r1 = plsc.load_gather(relv, [q_vec + (2 * w + 1) * NR])
                    return acc + a0 * r0 * b0v + a1 * r1 * b1v

                acc = lax.fori_loop(0, D // 2, wstep,
                                    jnp.zeros((L,), jnp.float32))
                emit(c * C + g * L, acc)

    def emit_fwd(off, acc):
        fwd_v[pl.ds(off, L)] = acc

    def emit_inv(off, acc):
        res = (fwd_v[pl.ds(off, L)] + acc) * 0.5
        res = jnp.minimum(jnp.maximum(res, -20.0), 20.0)
        out_v[pl.ds(off, L)] = res

    run_pass(rw, idx_h, idx_t, emit_fwd)
    run_pass(riw, idx_t, idx_h, emit_inv)

    pltpu.sync_copy(out_v, out.at[pl.ds(base, BPW)])


@jax.jit
def kernel(heads, rels, tails, ent_h, ent_t, rel_w, rel_inv_w):
    mesh = plsc.VectorSubcoreMesh(
        core_axis_name="c", subcore_axis_name="s",
        num_cores=NC, num_subcores=NS)
    f = pl.kernel(
        _body,
        out_type=jax.ShapeDtypeStruct((B,), jnp.float32),
        mesh=mesh,
        compiler_params=pltpu.CompilerParams(
            needs_layout_passes=False, use_tc_tiling_on_sc=True),
        scratch_types=[
            pltpu.VMEM((BPW,), jnp.int32),
            pltpu.VMEM((BPW,), jnp.int32),
            pltpu.VMEM((BPW,), jnp.int32),
            pltpu.VMEM((D * NR,), jnp.float32),
            pltpu.VMEM((C, 2 * D), jnp.int32),
            pltpu.VMEM((C, 2 * D), jnp.int32),
            pltpu.VMEM((C,), jnp.int32),
            pltpu.VMEM((C,), jnp.int32),
            pltpu.VMEM((BPW,), jnp.float32),
            pltpu.VMEM((BPW,), jnp.float32),
            pltpu.SemaphoreType.DMA,
        ],
    )
    def pack(t):
        b = lax.bitcast_convert_type(
            t.astype(jnp.bfloat16).reshape(NE, D // 2, 2), jnp.int32)
        return b.reshape(NE // 4, 2 * D)

    return f(heads.astype(jnp.int32), rels.astype(jnp.int32),
             tails.astype(jnp.int32), pack(ent_h), pack(ent_t),
             rel_w.T.reshape(D * NR), rel_inv_w.T.reshape(D * NR))


# speedup vs baseline: 2.6975x; 2.6975x over previous
import jax
import jax.numpy as jnp
from jax import lax
from jax.experimental import pallas as pl
from jax.experimental.pallas import tpu as pltpu
from jax.experimental.pallas import tpu_sc as plsc

B = 16384
D = 64
NE = 1000000
NR = 1000
NC = 2
NS = 16
L = 16
NW = NC * NS
BPW = B // NW
C = 128
NCHUNK = BPW // C
G = C // L

_MESH = dict(core_axis_name="c", subcore_axis_name="s",
             num_cores=NC, num_subcores=NS)
_PARAMS = dict(needs_layout_passes=False, use_tc_tiling_on_sc=True)


def _wid_base():
    wid = lax.axis_index("s") * NC + lax.axis_index("c")
    return wid * BPW


def _gather_body(ia, ib, tab, out_a, out_b, idx_a, idx_b, qa, qb, ba, bb,
                 sem):
    base = _wid_base()
    pltpu.sync_copy(ia.at[pl.ds(base, BPW)], idx_a)
    pltpu.sync_copy(ib.at[pl.ds(base, BPW)], idx_b)
    for c in range(NCHUNK):
        for g in range(G):
            va = idx_a[pl.ds(c * C + g * L, L)]
            vb = idx_b[pl.ds(c * C + g * L, L)]
            qa[pl.ds(g * L, L)] = va >> 1
            qb[pl.ds(g * L, L)] = vb >> 1
        cp0 = pltpu.async_copy(tab.at[qa], ba, sem)
        cp1 = pltpu.async_copy(tab.at[qb], bb, sem)
        cp0.wait()
        cp1.wait()
        pltpu.sync_copy(ba, out_a.at[pl.ds(base + c * C, C), :])
        pltpu.sync_copy(bb, out_b.at[pl.ds(base + c * C, C), :])


def _combine_body(heads, rels, tails, lhh, ltt, lht, lth, rw, riw, out,
                  idx_h, idx_r, idx_t, relv, b0, b1, fwd_v, out_v, sem):
    base = _wid_base()
    pltpu.sync_copy(heads.at[pl.ds(base, BPW)], idx_h)
    pltpu.sync_copy(rels.at[pl.ds(base, BPW)], idx_r)
    pltpu.sync_copy(tails.at[pl.ds(base, BPW)], idx_t)

    lane = lax.iota(jnp.int32, L)

    def run_pass(rel_flat, la, lb, ia_ref, ib_ref, emit):
        pltpu.sync_copy(rel_flat, relv)
        for c in range(NCHUNK):
            cp0 = pltpu.async_copy(la.at[pl.ds(base + c * C, C), :], b0, sem)
            cp1 = pltpu.async_copy(lb.at[pl.ds(base + c * C, C), :], b1, sem)
            cp0.wait()
            cp1.wait()

            for g in range(G):
                va = ia_ref[pl.ds(c * C + g * L, L)]
                vb = ib_ref[pl.ds(c * C + g * L, L)]
                rowv = lane + g * L
                offa = (va & 1) * D
                offb = (vb & 1) * D
                q_vec = idx_r[pl.ds(c * C + g * L, L)]

                def dstep(d, acc, rowv=rowv, offa=offa, offb=offb,
                          q_vec=q_vec):
                    a = plsc.load_gather(b0, [rowv, offa + d])
                    b = plsc.load_gather(b1, [rowv, offb + d])
                    r = plsc.load_gather(relv, [q_vec + d * NR])
                    return acc + a * r * b

                acc = lax.fori_loop(0, D, dstep, jnp.zeros((L,), jnp.float32))
                emit(c * C + g * L, acc)

    def emit_fwd(off, acc):
        fwd_v[pl.ds(off, L)] = acc

    def emit_inv(off, acc):
        res = (fwd_v[pl.ds(off, L)] + acc) * 0.5
        res = jnp.minimum(jnp.maximum(res, -20.0), 20.0)
        out_v[pl.ds(off, L)] = res

    run_pass(rw, lhh, ltt, idx_h, idx_t, emit_fwd)
    run_pass(riw, lht, lth, idx_t, idx_h, emit_inv)

    pltpu.sync_copy(out_v, out.at[pl.ds(base, BPW)])


@jax.jit
def kernel(heads, rels, tails, ent_h, ent_t, rel_w, rel_inv_w):
    mesh = plsc.VectorSubcoreMesh(**_MESH)
    heads = heads.astype(jnp.int32)
    rels = rels.astype(jnp.int32)
    tails = tails.astype(jnp.int32)

    gather = pl.kernel(
        _gather_body,
        out_type=(jax.ShapeDtypeStruct((B, 2 * D), jnp.float32),
                  jax.ShapeDtypeStruct((B, 2 * D), jnp.float32)),
        mesh=mesh,
        compiler_params=pltpu.CompilerParams(**_PARAMS),
        scratch_types=[
            pltpu.VMEM((BPW,), jnp.int32),
            pltpu.VMEM((BPW,), jnp.int32),
            pltpu.VMEM((C,), jnp.int32),
            pltpu.VMEM((C,), jnp.int32),
            pltpu.VMEM((C, 2 * D), jnp.float32),
            pltpu.VMEM((C, 2 * D), jnp.float32),
            pltpu.SemaphoreType.DMA,
        ],
    )
    lhh, lht = gather(heads, tails, ent_h.reshape(NE // 2, 2 * D))
    ltt, lth = gather(tails, heads, ent_t.reshape(NE // 2, 2 * D))

    combine = pl.kernel(
        _combine_body,
        out_type=jax.ShapeDtypeStruct((B,), jnp.float32),
        mesh=mesh,
        compiler_params=pltpu.CompilerParams(**_PARAMS),
        scratch_types=[
            pltpu.VMEM((BPW,), jnp.int32),
            pltpu.VMEM((BPW,), jnp.int32),
            pltpu.VMEM((BPW,), jnp.int32),
            pltpu.VMEM((D * NR,), jnp.float32),
            pltpu.VMEM((C, 2 * D), jnp.float32),
            pltpu.VMEM((C, 2 * D), jnp.float32),
            pltpu.VMEM((BPW,), jnp.float32),
            pltpu.VMEM((BPW,), jnp.float32),
            pltpu.SemaphoreType.DMA,
        ],
    )
    return combine(heads, rels, tails, lhh, ltt, lht, lth,
                   rel_w.T.reshape(D * NR), rel_inv_w.T.reshape(D * NR))


# speedup vs baseline: 2.7571x; 1.0221x over previous
import jax
import jax.numpy as jnp
from jax import lax
from jax.experimental import pallas as pl
from jax.experimental.pallas import tpu as pltpu
from jax.experimental.pallas import tpu_sc as plsc

B = 16384
D = 64
NE = 1000000
NR = 1000
NC = 2
NS = 16
L = 16
NW = NC * NS
BPW = B // NW
C = 128
NCHUNK = BPW // C
G = C // L


def _body(heads, rels, tails, eh, et, rw, riw, out,
          idx_h, idx_r, idx_t, relv, b0, b1, qa, qb, fwd_v, out_v, sem):
    cid = lax.axis_index("c")
    sid = lax.axis_index("s")
    wid = sid * NC + cid
    base = wid * BPW

    pltpu.sync_copy(heads.at[pl.ds(base, BPW)], idx_h)
    pltpu.sync_copy(rels.at[pl.ds(base, BPW)], idx_r)
    pltpu.sync_copy(tails.at[pl.ds(base, BPW)], idx_t)

    lane = lax.iota(jnp.int32, L)

    def run_pass(rel_flat, ia_ref, ib_ref, emit):
        pltpu.sync_copy(rel_flat, relv)

        for c in range(NCHUNK):
            for g in range(G):
                va = ia_ref[pl.ds(c * C + g * L, L)]
                vb = ib_ref[pl.ds(c * C + g * L, L)]
                qa[pl.ds(g * L, L)] = va >> 1
                qb[pl.ds(g * L, L)] = vb >> 1
            cp0 = pltpu.async_copy(eh.at[qa], b0, sem)
            cp1 = pltpu.async_copy(et.at[qb], b1, sem)
            cp0.wait()
            cp1.wait()

            for g in range(G):
                va = ia_ref[pl.ds(c * C + g * L, L)]
                vb = ib_ref[pl.ds(c * C + g * L, L)]
                rowv = lane + g * L
                offa = (va & 1) * D
                offb = (vb & 1) * D
                q_vec = idx_r[pl.ds(c * C + g * L, L)]

                def dstep(d, acc, rowv=rowv, offa=offa, offb=offb,
                          q_vec=q_vec):
                    a = plsc.load_gather(b0, [rowv, offa + d])
                    b = plsc.load_gather(b1, [rowv, offb + d])
                    r = plsc.load_gather(relv, [q_vec + d * NR])
                    return acc + a * r * b

                acc = lax.fori_loop(0, D, dstep, jnp.zeros((L,), jnp.float32))
                emit(c * C + g * L, acc)

    def emit_fwd(off, acc):
        fwd_v[pl.ds(off, L)] = acc

    def emit_inv(off, acc):
        res = (fwd_v[pl.ds(off, L)] + acc) * 0.5
        res = jnp.minimum(jnp.maximum(res, -20.0), 20.0)
        out_v[pl.ds(off, L)] = res

    run_pass(rw, idx_h, idx_t, emit_fwd)
    run_pass(riw, idx_t, idx_h, emit_inv)

    pltpu.sync_copy(out_v, out.at[pl.ds(base, BPW)])


@jax.jit
def kernel(heads, rels, tails, ent_h, ent_t, rel_w, rel_inv_w):
    mesh = plsc.VectorSubcoreMesh(
        core_axis_name="c", subcore_axis_name="s",
        num_cores=NC, num_subcores=NS)
    f = pl.kernel(
        _body,
        out_type=jax.ShapeDtypeStruct((B,), jnp.float32),
        mesh=mesh,
        compiler_params=pltpu.CompilerParams(
            needs_layout_passes=False, use_tc_tiling_on_sc=True),
        scratch_types=[
            pltpu.VMEM((BPW,), jnp.int32),
            pltpu.VMEM((BPW,), jnp.int32),
            pltpu.VMEM((BPW,), jnp.int32),
            pltpu.VMEM((D * NR,), jnp.float32),
            pltpu.VMEM((C, 2 * D), jnp.float32),
            pltpu.VMEM((C, 2 * D), jnp.float32),
            pltpu.VMEM((C,), jnp.int32),
            pltpu.VMEM((C,), jnp.int32),
            pltpu.VMEM((BPW,), jnp.float32),
            pltpu.VMEM((BPW,), jnp.float32),
            pltpu.SemaphoreType.DMA,
        ],
    )
    return f(heads.astype(jnp.int32), rels.astype(jnp.int32),
             tails.astype(jnp.int32),
             ent_h.reshape(NE // 2, 2 * D), ent_t.reshape(NE // 2, 2 * D),
             rel_w.T.reshape(D * NR), rel_inv_w.T.reshape(D * NR))
